# baseline (speedup 1.0000x reference)
import functools

import jax
import jax.numpy as jnp
from jax import lax
from jax.experimental import pallas as pl
from jax.experimental.pallas import tpu as pltpu
from jax.experimental.pallas import tpu_sc as plsc

F32 = jnp.float32


def _mv_kernel(w_ref, x_ref, b_ref, o_ref, *, relu):
    x2 = x_ref[...].reshape(1, -1)
    y = lax.dot_general(w_ref[...], x2, (((1,), (1,)), ((), ())),
                        preferred_element_type=F32)
    y = y.reshape(-1) + b_ref[...]
    o_ref[...] = jnp.maximum(y, 0.0) if relu else y


def _matvec(w, x, b, blk, relu=False):
    n, d = w.shape
    return pl.pallas_call(
        functools.partial(_mv_kernel, relu=relu),
        grid=(n // blk,),
        in_specs=[
            pl.BlockSpec((blk, d), lambda i: (i, 0)),
            pl.BlockSpec((d,), lambda i: (0,)),
            pl.BlockSpec((blk,), lambda i: (i,)),
        ],
        out_specs=pl.BlockSpec((blk,), lambda i: (i,)),
        out_shape=jax.ShapeDtypeStruct((n,), F32),
    )(w, x, b)


def _sims_kernel(p_ref, k_ref, o_ref):
    p = p_ref[...]
    kv = k_ref[...].reshape(1, -1)
    dot = lax.dot_general(p, kv, (((1,), (1,)), ((), ())),
                          preferred_element_type=F32)
    sq = jnp.sum(p * p, axis=1)
    nk = jnp.sqrt(jnp.sum(kv * kv))
    denom = jnp.maximum(jnp.sqrt(sq), 1e-12) * jnp.maximum(nk, 1e-12)
    o_ref[...] = dot.reshape(-1) / denom


def _sims(protos, k, blk):
    n, d = protos.shape
    return pl.pallas_call(
        _sims_kernel,
        grid=(n // blk,),
        in_specs=[
            pl.BlockSpec((blk, d), lambda i: (i, 0)),
            pl.BlockSpec((d,), lambda i: (0,)),
        ],
        out_specs=pl.BlockSpec((blk,), lambda i: (i,)),
        out_shape=jax.ShapeDtypeStruct((n,), F32),
    )(protos, k)


def _sc_argmax(sims):
    n = sims.shape[0]
    nw = 32
    per_w = n // nw
    nchunks = per_w // 16
    mesh = plsc.VectorSubcoreMesh(core_axis_name="c", subcore_axis_name="s",
                                  num_cores=2, num_subcores=16)

    @functools.partial(
        pl.kernel,
        out_type=[jax.ShapeDtypeStruct((nw, 16), F32),
                  jax.ShapeDtypeStruct((nw, 16), jnp.int32)],
        mesh=mesh,
        compiler_params=pltpu.CompilerParams(use_tc_tiling_on_sc=False),
        scratch_types=[
            pltpu.VMEM((per_w,), F32),
            pltpu.VMEM((16,), F32),
            pltpu.VMEM((16,), jnp.int32),
        ],
    )
    def body(sims_hbm, best_out, idx_out, sims_v, bv_v, bi_v):
        cid = lax.axis_index("c")
        sid = lax.axis_index("s")
        wid = sid * 2 + cid
        base = wid * per_w
        pltpu.sync_copy(sims_hbm.at[pl.ds(base, per_w)], sims_v)

        def chunk_body(j, carry):
            m, mi = carry
            v = sims_v[pl.ds(j * 16, 16)]
            upd = v > m
            return jnp.where(upd, v, m), jnp.where(upd, j, mi)

        m0 = jnp.full((16,), -3.0e38, F32)
        mi0 = jnp.zeros((16,), jnp.int32)
        m, mi = lax.fori_loop(0, nchunks, chunk_body, (m0, mi0))

        lanes = jnp.arange(16, dtype=jnp.int32)
        gi = base + mi * 16 + lanes
        dn = lax.GatherDimensionNumbers(
            offset_dims=(), collapsed_slice_dims=(0,),
            start_index_map=(0,))

        def permute(v, p):
            return lax.gather(
                v, p[:, None], dn, (1,),
                mode=lax.GatherScatterMode.PROMISE_IN_BOUNDS)

        for shift in (1, 2, 4, 8):
            p = lanes ^ shift
            m2 = permute(m, p)
            gi2 = permute(gi, p)
            upd = (m2 > m) | ((m2 == m) & (gi2 < gi))
            m = jnp.where(upd, m2, m)
            gi = jnp.where(upd, gi2, gi)

        bv_v[...] = m
        bi_v[...] = gi
        pltpu.sync_copy(bv_v, best_out.at[wid])
        pltpu.sync_copy(bi_v, idx_out.at[wid])

    return body(sims)


def _final_kernel(b_ref, i_ref, memt_ref, wpt_ref, bp_ref, pfc_ref,
                  wnt_ref, bn_ref, g1_ref, g1b_ref, g2_ref, g2b_ref, td_ref,
                  pfcnew_ref, nm_ref, alpha_ref, tile_v, row_v, sem):
    step = pl.program_id(0)
    bests = b_ref[:, 0:1]
    idxs = i_ref[:, 0:1]
    best = jnp.max(bests)
    widx = jnp.min(jnp.where(bests == best, idxs, jnp.int32(2**31 - 1)))

    @pl.when(step == 0)
    def _():
        start = pl.multiple_of((widx // 128) * 128, 128)
        cp = pltpu.make_async_copy(memt_ref.at[:, pl.ds(start, 128)],
                                   tile_v, sem)
        cp.start()
        cp.wait()
        lane = widx % 128
        mask = lax.broadcasted_iota(jnp.int32, (1, 128), 1) == lane
        row_v[...] = jnp.sum(jnp.where(mask, tile_v[...], 0.0),
                             axis=1, keepdims=True)

    ro = row_v[...]
    td = td_ref[0]
    hg = jnp.tanh(g1_ref[:, 0] * best + g1_ref[:, 1] * td + g1b_ref[...])
    alpha = jnp.tanh(jnp.sum(g2_ref[0, :] * hg) + g2b_ref[0])
    delta = jnp.sum(wpt_ref[...] * ro, axis=0, keepdims=True)
    delta = delta.reshape(-1) + bp_ref[...]
    pfcnew_ref[...] = pfc_ref[...] + alpha * delta

    @pl.when(step == pl.num_programs(0) - 1)
    def _():
        nm = jnp.sum(wnt_ref[...] * ro, axis=0, keepdims=True).reshape(-1)
        nm = nm + bn_ref[...]
        col = lax.iota(jnp.int32, nm.shape[0])
        hi = jnp.where(col < 32, 1.0, 0.5)
        nm_ref[...] = jnp.clip(nm, 0.1, hi)
        alpha_ref[...] = jnp.reshape(alpha, (1,))


def _final(bests, idxs, memt, wpt, bp, pfc, wnt, bn,
           g1, g1b, g2, g2b, td, blk):
    d, n = wpt.shape
    nmo = wnt.shape[1]
    zero2 = lambda i: (0, 0)
    zero1 = lambda i: (0,)
    return pl.pallas_call(
        _final_kernel,
        grid=(n // blk,),
        in_specs=[
            pl.BlockSpec(bests.shape, zero2),
            pl.BlockSpec(idxs.shape, zero2),
            pl.BlockSpec(memory_space=pltpu.MemorySpace.HBM),
            pl.BlockSpec((d, blk), lambda i: (0, i)),
            pl.BlockSpec((blk,), lambda i: (i,)),
            pl.BlockSpec((blk,), lambda i: (i,)),
            pl.BlockSpec(wnt.shape, zero2),
            pl.BlockSpec((nmo,), zero1),
            pl.BlockSpec(g1.shape, zero2),
            pl.BlockSpec((g1.shape[0],), zero1),
            pl.BlockSpec(g2.shape, zero2),
            pl.BlockSpec((1,), zero1),
            pl.BlockSpec((1,), zero1),
        ],
        out_specs=[
            pl.BlockSpec((blk,), lambda i: (i,)),
            pl.BlockSpec((nmo,), zero1),
            pl.BlockSpec((1,), zero1),
        ],
        out_shape=[
            jax.ShapeDtypeStruct((n,), F32),
            jax.ShapeDtypeStruct((nmo,), F32),
            jax.ShapeDtypeStruct((1,), F32),
        ],
        scratch_shapes=[
            pltpu.VMEM((d, 128), F32),
            pltpu.VMEM((d, 1), F32),
            pltpu.SemaphoreType.DMA,
        ],
    )(bests, idxs, memt, wpt, bp, pfc, wnt, bn,
      g1, g1b, g2, g2b, td)


def kernel(activation_summary, pfc_state, current_td_error, prototypes,
           W1, b1, W2, b2, memory, G1, g1, G2, g2, Wp, bp, Wn, bn):
    x = jnp.concatenate([activation_summary, pfc_state[0]])

    h = _matvec(W1, x, b1, blk=256, relu=True)
    k = _matvec(W2, h, b2, blk=256)
    sims = _sims(prototypes, k, blk=256)

    bests, idxs = _sc_argmax(sims)
    td = jnp.abs(jnp.asarray(current_td_error)).astype(F32).reshape(1)

    pfc_new, nm, alpha = _final(
        bests, idxs, memory.T, Wp.T, bp, pfc_state[0],
        Wn.T, bn, G1, g1, G2, g2, td, blk=1024)

    return jnp.concatenate([pfc_new, nm, alpha])


# speedup vs baseline: 1.0489x; 1.0489x over previous
import functools

import jax
import jax.numpy as jnp
from jax import lax
from jax.experimental import pallas as pl
from jax.experimental.pallas import tpu as pltpu
from jax.experimental.pallas import tpu_sc as plsc

F32 = jnp.float32


def _mv_kernel(w_ref, x_ref, b_ref, o_ref, *, relu):
    x2 = x_ref[...].reshape(1, -1)
    y = lax.dot_general(w_ref[...], x2, (((1,), (1,)), ((), ())),
                        preferred_element_type=F32)
    y = y.reshape(-1) + b_ref[...]
    o_ref[...] = jnp.maximum(y, 0.0) if relu else y


def _matvec(w, x, b, blk, relu=False):
    n, d = w.shape
    return pl.pallas_call(
        functools.partial(_mv_kernel, relu=relu),
        grid=(n // blk,),
        in_specs=[
            pl.BlockSpec((blk, d), lambda i: (i, 0)),
            pl.BlockSpec((d,), lambda i: (0,)),
            pl.BlockSpec((blk,), lambda i: (i,)),
        ],
        out_specs=pl.BlockSpec((blk,), lambda i: (i,)),
        out_shape=jax.ShapeDtypeStruct((n,), F32),
    )(w, x, b)


def _sims_kernel(p_ref, k_ref, o_ref):
    p = p_ref[...]
    kv = k_ref[...].reshape(1, -1)
    dot = lax.dot_general(p, kv, (((1,), (1,)), ((), ())),
                          preferred_element_type=F32)
    sq = jnp.sum(p * p, axis=1)
    nk = jnp.sqrt(jnp.sum(kv * kv))
    denom = jnp.maximum(jnp.sqrt(sq), 1e-12) * jnp.maximum(nk, 1e-12)
    o_ref[...] = dot.reshape(-1) / denom


def _sims(protos, k, blk):
    n, d = protos.shape
    return pl.pallas_call(
        _sims_kernel,
        grid=(n // blk,),
        in_specs=[
            pl.BlockSpec((blk, d), lambda i: (i, 0)),
            pl.BlockSpec((d,), lambda i: (0,)),
        ],
        out_specs=pl.BlockSpec((blk,), lambda i: (i,)),
        out_shape=jax.ShapeDtypeStruct((n,), F32),
    )(protos, k)


def _sc_argmax(sims):
    n = sims.shape[0]
    nw = 32
    per_w = n // nw
    nchunks = per_w // 16
    mesh = plsc.VectorSubcoreMesh(core_axis_name="c", subcore_axis_name="s",
                                  num_cores=2, num_subcores=16)

    @functools.partial(
        pl.kernel,
        out_type=[jax.ShapeDtypeStruct((nw, 16), F32),
                  jax.ShapeDtypeStruct((nw, 16), jnp.int32)],
        mesh=mesh,
        compiler_params=pltpu.CompilerParams(use_tc_tiling_on_sc=False),
        scratch_types=[
            pltpu.VMEM((per_w,), F32),
            pltpu.VMEM((16,), F32),
            pltpu.VMEM((16,), jnp.int32),
        ],
    )
    def body(sims_hbm, best_out, idx_out, sims_v, bv_v, bi_v):
        cid = lax.axis_index("c")
        sid = lax.axis_index("s")
        wid = sid * 2 + cid
        base = wid * per_w
        pltpu.sync_copy(sims_hbm.at[pl.ds(base, per_w)], sims_v)

        def chunk_body(j, carry):
            m, mi = carry
            v = sims_v[pl.ds(j * 16, 16)]
            upd = v > m
            return jnp.where(upd, v, m), jnp.where(upd, j, mi)

        m0 = jnp.full((16,), -3.0e38, F32)
        mi0 = jnp.zeros((16,), jnp.int32)
        m, mi = lax.fori_loop(0, nchunks, chunk_body, (m0, mi0))

        lanes = jnp.arange(16, dtype=jnp.int32)
        gi = base + mi * 16 + lanes
        dn = lax.GatherDimensionNumbers(
            offset_dims=(), collapsed_slice_dims=(0,),
            start_index_map=(0,))

        def permute(v, p):
            return lax.gather(
                v, p[:, None], dn, (1,),
                mode=lax.GatherScatterMode.PROMISE_IN_BOUNDS)

        for shift in (1, 2, 4, 8):
            p = lanes ^ shift
            m2 = permute(m, p)
            gi2 = permute(gi, p)
            upd = (m2 > m) | ((m2 == m) & (gi2 < gi))
            m = jnp.where(upd, m2, m)
            gi = jnp.where(upd, gi2, gi)

        bv_v[...] = m
        bi_v[...] = gi
        pltpu.sync_copy(bv_v, best_out.at[wid])
        pltpu.sync_copy(bi_v, idx_out.at[wid])

    return body(sims)


def _final_kernel(b_ref, i_ref, memt_ref, wpt_ref, bp_ref, pfc_ref,
                  wnt_ref, bn_ref, g1_ref, g1b_ref, g2_ref, g2b_ref, td_ref,
                  pfcnew_ref, nm_ref, alpha_ref, alpha_sm, tile_v, row_v,
                  sem):
    step = pl.program_id(0)

    @pl.when(step == 0)
    def _():
        bests = b_ref[:, 0:1]
        idxs = i_ref[:, 0:1]
        best = jnp.max(bests)
        widx = jnp.min(jnp.where(bests == best, idxs,
                                 jnp.int32(2**31 - 1)))
        start = pl.multiple_of((widx // 128) * 128, 128)
        cp = pltpu.make_async_copy(memt_ref.at[:, pl.ds(start, 128)],
                                   tile_v, sem)
        cp.start()
        td = td_ref[0]
        hg = jnp.tanh(g1_ref[:, 0] * best + g1_ref[:, 1] * td
                      + g1b_ref[...])
        alpha_sm[0] = jnp.tanh(jnp.sum(g2_ref[0, :] * hg) + g2b_ref[0])
        cp.wait()
        lane = widx % 128
        mask = lax.broadcasted_iota(jnp.int32, (1, 128), 1) == lane
        row_v[...] = jnp.sum(jnp.where(mask, tile_v[...], 0.0),
                             axis=1, keepdims=True)

    ro = row_v[...]
    alpha = alpha_sm[0]
    delta = jnp.sum(wpt_ref[...] * ro, axis=0, keepdims=True)
    delta = delta.reshape(-1) + bp_ref[...]
    pfcnew_ref[...] = pfc_ref[...] + alpha * delta

    @pl.when(step == pl.num_programs(0) - 1)
    def _():
        nm = jnp.sum(wnt_ref[...] * ro, axis=0, keepdims=True).reshape(-1)
        nm = nm + bn_ref[...]
        col = lax.iota(jnp.int32, nm.shape[0])
        hi = jnp.where(col < 32, 1.0, 0.5)
        nm_ref[...] = jnp.clip(nm, 0.1, hi)
        alpha_ref[...] = jnp.reshape(alpha, (1,))


def _final(bests, idxs, memt, wpt, bp, pfc, wnt, bn,
           g1, g1b, g2, g2b, td, blk):
    d, n = wpt.shape
    nmo = wnt.shape[1]
    zero2 = lambda i: (0, 0)
    zero1 = lambda i: (0,)
    return pl.pallas_call(
        _final_kernel,
        grid=(n // blk,),
        in_specs=[
            pl.BlockSpec(bests.shape, zero2),
            pl.BlockSpec(idxs.shape, zero2),
            pl.BlockSpec(memory_space=pltpu.MemorySpace.HBM),
            pl.BlockSpec((d, blk), lambda i: (0, i)),
            pl.BlockSpec((blk,), lambda i: (i,)),
            pl.BlockSpec((blk,), lambda i: (i,)),
            pl.BlockSpec(wnt.shape, zero2),
            pl.BlockSpec((nmo,), zero1),
            pl.BlockSpec(g1.shape, zero2),
            pl.BlockSpec((g1.shape[0],), zero1),
            pl.BlockSpec(g2.shape, zero2),
            pl.BlockSpec((1,), zero1),
            pl.BlockSpec((1,), zero1),
        ],
        out_specs=[
            pl.BlockSpec((blk,), lambda i: (i,)),
            pl.BlockSpec((nmo,), zero1),
            pl.BlockSpec((1,), zero1),
        ],
        out_shape=[
            jax.ShapeDtypeStruct((n,), F32),
            jax.ShapeDtypeStruct((nmo,), F32),
            jax.ShapeDtypeStruct((1,), F32),
        ],
        scratch_shapes=[
            pltpu.SMEM((1,), F32),
            pltpu.VMEM((d, 128), F32),
            pltpu.VMEM((d, 1), F32),
            pltpu.SemaphoreType.DMA,
        ],
    )(bests, idxs, memt, wpt, bp, pfc, wnt, bn,
      g1, g1b, g2, g2b, td)


def kernel(activation_summary, pfc_state, current_td_error, prototypes,
           W1, b1, W2, b2, memory, G1, g1, G2, g2, Wp, bp, Wn, bn):
    x = jnp.concatenate([activation_summary, pfc_state[0]])

    h = _matvec(W1, x, b1, blk=256, relu=True)
    k = _matvec(W2, h, b2, blk=256)
    sims = _sims(prototypes, k, blk=512)

    bests, idxs = _sc_argmax(sims)
    td = jnp.abs(jnp.asarray(current_td_error)).astype(F32).reshape(1)

    pfc_new, nm, alpha = _final(
        bests, idxs, memory.T, Wp.T, bp, pfc_state[0],
        Wn.T, bn, G1, g1, G2, g2, td, blk=1024)

    return jnp.concatenate([pfc_new, nm, alpha])


# speedup vs baseline: 1.0504x; 1.0014x over previous
import functools

import jax
import jax.numpy as jnp
from jax import lax
from jax.experimental import pallas as pl
from jax.experimental.pallas import tpu as pltpu
from jax.experimental.pallas import tpu_sc as plsc

F32 = jnp.float32


def _mv_kernel(w_ref, x_ref, b_ref, o_ref, *, relu):
    x2 = x_ref[...].reshape(1, -1)
    y = lax.dot_general(w_ref[...], x2, (((1,), (1,)), ((), ())),
                        preferred_element_type=F32)
    y = y.reshape(-1) + b_ref[...]
    o_ref[...] = jnp.maximum(y, 0.0) if relu else y


def _matvec(w, x, b, blk, relu=False):
    n, d = w.shape
    return pl.pallas_call(
        functools.partial(_mv_kernel, relu=relu),
        grid=(n // blk,),
        in_specs=[
            pl.BlockSpec((blk, d), lambda i: (i, 0)),
            pl.BlockSpec((d,), lambda i: (0,)),
            pl.BlockSpec((blk,), lambda i: (i,)),
        ],
        out_specs=pl.BlockSpec((blk,), lambda i: (i,)),
        out_shape=jax.ShapeDtypeStruct((n,), F32),
    )(w, x, b)


def _sims_kernel(p_ref, k_ref, o_ref):
    p = p_ref[...]
    kv = k_ref[...].reshape(1, -1)
    dot = lax.dot_general(p, kv, (((1,), (1,)), ((), ())),
                          preferred_element_type=F32)
    sq = jnp.sum(p * p, axis=1)
    nk = jnp.sqrt(jnp.sum(kv * kv))
    denom = jnp.maximum(jnp.sqrt(sq), 1e-12) * jnp.maximum(nk, 1e-12)
    o_ref[...] = dot.reshape(-1) / denom


def _sims(protos, k, blk):
    n, d = protos.shape
    return pl.pallas_call(
        _sims_kernel,
        grid=(n // blk,),
        in_specs=[
            pl.BlockSpec((blk, d), lambda i: (i, 0)),
            pl.BlockSpec((d,), lambda i: (0,)),
        ],
        out_specs=pl.BlockSpec((blk,), lambda i: (i,)),
        out_shape=jax.ShapeDtypeStruct((n,), F32),
    )(protos, k)


def _sc_argmax(sims):
    n = sims.shape[0]
    nw = 32
    per_w = n // nw
    nchunks = per_w // 16
    mesh = plsc.VectorSubcoreMesh(core_axis_name="c", subcore_axis_name="s",
                                  num_cores=2, num_subcores=16)

    @functools.partial(
        pl.kernel,
        out_type=[jax.ShapeDtypeStruct((nw, 16), F32),
                  jax.ShapeDtypeStruct((nw, 16), jnp.int32)],
        mesh=mesh,
        compiler_params=pltpu.CompilerParams(use_tc_tiling_on_sc=False),
        scratch_types=[
            pltpu.VMEM((per_w,), F32),
            pltpu.VMEM((16,), F32),
            pltpu.VMEM((16,), jnp.int32),
        ],
    )
    def body(sims_hbm, best_out, idx_out, sims_v, bv_v, bi_v):
        cid = lax.axis_index("c")
        sid = lax.axis_index("s")
        wid = sid * 2 + cid
        base = wid * per_w
        pltpu.sync_copy(sims_hbm.at[pl.ds(base, per_w)], sims_v)

        def chunk_body(j, carry):
            m, mi = carry
            v = sims_v[pl.ds(j * 16, 16)]
            upd = v > m
            return jnp.where(upd, v, m), jnp.where(upd, j, mi)

        m0 = jnp.full((16,), -3.0e38, F32)
        mi0 = jnp.zeros((16,), jnp.int32)
        m, mi = lax.fori_loop(0, nchunks, chunk_body, (m0, mi0))

        lanes = jnp.arange(16, dtype=jnp.int32)
        gi = base + mi * 16 + lanes
        dn = lax.GatherDimensionNumbers(
            offset_dims=(), collapsed_slice_dims=(0,),
            start_index_map=(0,))

        def permute(v, p):
            return lax.gather(
                v, p[:, None], dn, (1,),
                mode=lax.GatherScatterMode.PROMISE_IN_BOUNDS)

        for shift in (1, 2, 4, 8):
            p = lanes ^ shift
            m2 = permute(m, p)
            gi2 = permute(gi, p)
            upd = (m2 > m) | ((m2 == m) & (gi2 < gi))
            m = jnp.where(upd, m2, m)
            gi = jnp.where(upd, gi2, gi)

        bv_v[...] = m
        bi_v[...] = gi
        pltpu.sync_copy(bv_v, best_out.at[wid])
        pltpu.sync_copy(bi_v, idx_out.at[wid])

    return body(sims)


def _final_kernel(b_ref, i_ref, memt_ref, wpt_ref, bp_ref, pfc_ref,
                  wn_ref, bn_ref, g1_ref, g1b_ref, g2_ref, g2b_ref, td_ref,
                  pfcnew_ref, nm_ref, alpha_ref, alpha_sm, b_v, i_v,
                  tile_v, row_v, sem):
    step = pl.program_id(0)

    @pl.when(step == 0)
    def _():
        cb = pltpu.make_async_copy(b_ref, b_v, sem)
        cb.start()
        cb.wait()
        ci = pltpu.make_async_copy(i_ref, i_v, sem)
        ci.start()
        ci.wait()
        bests = b_v[:, 0:1]
        idxs = i_v[:, 0:1]
        best = jnp.max(bests)
        widx = jnp.min(jnp.where(bests == best, idxs,
                                 jnp.int32(2**31 - 1)))
        start = pl.multiple_of((widx // 128) * 128, 128)
        cp = pltpu.make_async_copy(memt_ref.at[:, pl.ds(start, 128)],
                                   tile_v, sem)
        cp.start()
        td = td_ref[0]
        hg = jnp.tanh(g1_ref[:, 0] * best + g1_ref[:, 1] * td
                      + g1b_ref[...])
        alpha_sm[0] = jnp.tanh(jnp.sum(g2_ref[0, :] * hg) + g2b_ref[0])
        cp.wait()
        lane = widx % 128
        mask = lax.broadcasted_iota(jnp.int32, (1, 128), 1) == lane
        row_v[...] = jnp.sum(jnp.where(mask, tile_v[...], 0.0),
                             axis=1, keepdims=True)

    ro = row_v[...]
    alpha = alpha_sm[0]
    delta = jnp.sum(wpt_ref[...] * ro, axis=0, keepdims=True)
    delta = delta.reshape(-1) + bp_ref[...]
    pfcnew_ref[...] = pfc_ref[...] + alpha * delta

    @pl.when(step == pl.num_programs(0) - 1)
    def _():
        nm = lax.dot_general(wn_ref[...], ro, (((1,), (0,)), ((), ())),
                             preferred_element_type=F32)
        nm = nm.reshape(-1) + bn_ref[...]
        col = lax.iota(jnp.int32, nm.shape[0])
        hi = jnp.where(col < 32, 1.0, 0.5)
        nm_ref[...] = jnp.clip(nm, 0.1, hi)
        alpha_ref[...] = jnp.reshape(alpha, (1,))


def _final(bests, idxs, memt, wpt, bp, pfc, wn, bn,
           g1, g1b, g2, g2b, td, blk):
    d, n = wpt.shape
    nmo = wn.shape[0]
    zero2 = lambda i: (0, 0)
    zero1 = lambda i: (0,)
    return pl.pallas_call(
        _final_kernel,
        grid=(n // blk,),
        in_specs=[
            pl.BlockSpec(memory_space=pltpu.MemorySpace.HBM),
            pl.BlockSpec(memory_space=pltpu.MemorySpace.HBM),
            pl.BlockSpec(memory_space=pltpu.MemorySpace.HBM),
            pl.BlockSpec((d, blk), lambda i: (0, i)),
            pl.BlockSpec((blk,), lambda i: (i,)),
            pl.BlockSpec((blk,), lambda i: (i,)),
            pl.BlockSpec(wn.shape, zero2),
            pl.BlockSpec((nmo,), zero1),
            pl.BlockSpec(g1.shape, zero2),
            pl.BlockSpec((g1.shape[0],), zero1),
            pl.BlockSpec(g2.shape, zero2),
            pl.BlockSpec((1,), zero1),
            pl.BlockSpec((1,), zero1),
        ],
        out_specs=[
            pl.BlockSpec((blk,), lambda i: (i,)),
            pl.BlockSpec((nmo,), zero1),
            pl.BlockSpec((1,), zero1),
        ],
        out_shape=[
            jax.ShapeDtypeStruct((n,), F32),
            jax.ShapeDtypeStruct((nmo,), F32),
            jax.ShapeDtypeStruct((1,), F32),
        ],
        scratch_shapes=[
            pltpu.SMEM((1,), F32),
            pltpu.VMEM(bests.shape, F32),
            pltpu.VMEM(idxs.shape, jnp.int32),
            pltpu.VMEM((d, 128), F32),
            pltpu.VMEM((d, 1), F32),
            pltpu.SemaphoreType.DMA,
        ],
    )(bests, idxs, memt, wpt, bp, pfc, wn, bn,
      g1, g1b, g2, g2b, td)


def kernel(activation_summary, pfc_state, current_td_error, prototypes,
           W1, b1, W2, b2, memory, G1, g1, G2, g2, Wp, bp, Wn, bn):
    x = jnp.concatenate([activation_summary, pfc_state[0]])

    h = _matvec(W1, x, b1, blk=256, relu=True)
    k = _matvec(W2, h, b2, blk=256)
    sims = _sims(prototypes, k, blk=512)

    bests, idxs = _sc_argmax(sims)
    td = jnp.abs(jnp.asarray(current_td_error)).astype(F32).reshape(1)

    pfc_new, nm, alpha = _final(
        bests, idxs, memory.T, Wp.T, bp, pfc_state[0],
        Wn, bn, G1, g1, G2, g2, td, blk=1024)

    return jnp.concatenate([pfc_new, nm, alpha])
